# baseline (device time: 41474 ns/iter reference)
import jax
import jax.numpy as jnp
from jax import lax
from jax.experimental import pallas as pl
from jax.experimental.pallas import tpu as pltpu

M = 1024
D = 1024
HALF = M // 2
K = 4
CH = HALF // K
EPS = 1e-6


def kernel(partial, resid, gamma):
    x2d = partial.reshape(M, D)
    g2d = gamma.reshape(1, D)

    def body(x_ref, r_ref, g_ref, out_ref, recv_ref, s1, r1, s2, r2):
        my_x = lax.axis_index("x")
        my_y = lax.axis_index("y")
        h = (my_x + my_y) % 2
        off_mine = h * HALF
        off_other = (1 - h) * HALF

        xnbr = (1 - my_x, my_y)
        ynbr = (my_x, 1 - my_y)

        barrier = pltpu.get_barrier_semaphore()
        for nbr in (xnbr, ynbr):
            pl.semaphore_signal(
                barrier, inc=1, device_id=nbr,
                device_id_type=pl.DeviceIdType.MESH,
            )
        pl.semaphore_wait(barrier, 2)

        p1 = []
        for c in range(K):
            rdma = pltpu.make_async_remote_copy(
                src_ref=x_ref.at[pl.ds(off_other + c * CH, CH), :],
                dst_ref=recv_ref.at[pl.ds(c * CH, CH), :],
                send_sem=s1.at[c],
                recv_sem=r1.at[c],
                device_id=xnbr,
                device_id_type=pl.DeviceIdType.MESH,
            )
            rdma.start()
            p1.append(rdma)

        p2 = []
        for c in range(K):
            p1[c].wait_recv()
            row = off_mine + c * CH
            ysum = (
                x_ref[pl.ds(row, CH), :]
                + recv_ref[pl.ds(c * CH, CH), :]
                + r_ref[pl.ds(row, CH), :]
            )
            ms = jnp.mean(ysum * ysum, axis=-1, keepdims=True)
            out_ref[pl.ds(row, CH), :] = (
                ysum * lax.rsqrt(ms + EPS) * g_ref[...]
            )
            rdma2 = pltpu.make_async_remote_copy(
                src_ref=out_ref.at[pl.ds(row, CH), :],
                dst_ref=out_ref.at[pl.ds(row, CH), :],
                send_sem=s2.at[c],
                recv_sem=r2.at[c],
                device_id=ynbr,
                device_id_type=pl.DeviceIdType.MESH,
            )
            rdma2.start()
            p2.append(rdma2)

        for c in range(K):
            p2[c].wait_recv()
            p1[c].wait_send()
            p2[c].wait_send()

    return pl.pallas_call(
        body,
        out_shape=jax.ShapeDtypeStruct((M, D), jnp.float32),
        in_specs=[pl.BlockSpec(memory_space=pltpu.VMEM)] * 3,
        out_specs=pl.BlockSpec(memory_space=pltpu.VMEM),
        scratch_shapes=[
            pltpu.VMEM((HALF, D), jnp.float32),
            pltpu.SemaphoreType.DMA((K,)),
            pltpu.SemaphoreType.DMA((K,)),
            pltpu.SemaphoreType.DMA((K,)),
            pltpu.SemaphoreType.DMA((K,)),
        ],
        compiler_params=pltpu.CompilerParams(collective_id=0),
    )(x2d, resid, g2d)


# device time: 38791 ns/iter; 1.0692x vs baseline; 1.0692x over previous
import jax
import jax.numpy as jnp
from jax import lax
from jax.experimental import pallas as pl
from jax.experimental.pallas import tpu as pltpu

M = 1024
D = 1024
HALF = M // 2
K = 8
CH = HALF // K
EPS = 1e-6


def kernel(partial, resid, gamma):
    x2d = partial.reshape(M, D)
    g2d = gamma.reshape(1, D)

    def body(x_ref, r_ref, g_ref, out_ref, recv_ref, s1, r1, s2, r2):
        my_x = lax.axis_index("x")
        my_y = lax.axis_index("y")
        h = (my_x + my_y) % 2
        off_mine = h * HALF
        off_other = (1 - h) * HALF

        xnbr = (1 - my_x, my_y)
        ynbr = (my_x, 1 - my_y)

        barrier = pltpu.get_barrier_semaphore()
        for nbr in (xnbr, ynbr):
            pl.semaphore_signal(
                barrier, inc=1, device_id=nbr,
                device_id_type=pl.DeviceIdType.MESH,
            )
        pl.semaphore_wait(barrier, 2)

        p1 = []
        for c in range(K):
            rdma = pltpu.make_async_remote_copy(
                src_ref=x_ref.at[pl.ds(off_other + c * CH, CH), :],
                dst_ref=recv_ref.at[pl.ds(c * CH, CH), :],
                send_sem=s1.at[c],
                recv_sem=r1.at[c],
                device_id=xnbr,
                device_id_type=pl.DeviceIdType.MESH,
            )
            rdma.start()
            p1.append(rdma)

        p2 = []
        for c in range(K):
            p1[c].wait_recv()
            row = off_mine + c * CH
            ysum = (
                x_ref[pl.ds(row, CH), :]
                + recv_ref[pl.ds(c * CH, CH), :]
                + r_ref[pl.ds(row, CH), :]
            )
            ms = jnp.mean(ysum * ysum, axis=-1, keepdims=True)
            out_ref[pl.ds(row, CH), :] = (
                ysum * lax.rsqrt(ms + EPS) * g_ref[...]
            )
            rdma2 = pltpu.make_async_remote_copy(
                src_ref=out_ref.at[pl.ds(row, CH), :],
                dst_ref=out_ref.at[pl.ds(row, CH), :],
                send_sem=s2.at[c],
                recv_sem=r2.at[c],
                device_id=ynbr,
                device_id_type=pl.DeviceIdType.MESH,
            )
            rdma2.start()
            p2.append(rdma2)

        for c in range(K):
            p2[c].wait_recv()
            p1[c].wait_send()
            p2[c].wait_send()

    return pl.pallas_call(
        body,
        out_shape=jax.ShapeDtypeStruct((M, D), jnp.float32),
        in_specs=[pl.BlockSpec(memory_space=pltpu.VMEM)] * 3,
        out_specs=pl.BlockSpec(memory_space=pltpu.VMEM),
        scratch_shapes=[
            pltpu.VMEM((HALF, D), jnp.float32),
            pltpu.SemaphoreType.DMA((K,)),
            pltpu.SemaphoreType.DMA((K,)),
            pltpu.SemaphoreType.DMA((K,)),
            pltpu.SemaphoreType.DMA((K,)),
        ],
        compiler_params=pltpu.CompilerParams(collective_id=0),
    )(x2d, resid, g2d)
